# baseline (speedup 1.0000x reference)
import jax
import jax.numpy as jnp
from jax import lax
from jax.experimental import pallas as pl
from jax.experimental.pallas import tpu as pltpu
from jax.experimental.pallas import tpu_sc as plsc

NC = 2
NS = 16
LANES = 16


def _make_sc_trow(H, T):
    mesh = plsc.VectorSubcoreMesh(core_axis_name="c", subcore_axis_name="s",
                                  num_cores=NC, num_subcores=NS)

    def body(t_hbm, tidx_hbm, out_hbm, idx_v, trow_v, sem):
        wid = lax.axis_index("s") * NC + lax.axis_index("c")

        @pl.when(wid == 0)
        def _():
            pltpu.sync_copy(tidx_hbm, idx_v)
            pltpu.async_copy(t_hbm.at[idx_v], trow_v, sem).wait()
            pltpu.sync_copy(trow_v, out_hbm)

    return pl.kernel(
        body,
        out_type=jax.ShapeDtypeStruct((LANES, H), jnp.float32),
        mesh=mesh,
        compiler_params=pltpu.CompilerParams(use_tc_tiling_on_sc=False,
                                             needs_layout_passes=False,
                                             skip_device_barrier=True),
        scratch_types=[
            pltpu.VMEM((LANES,), jnp.int32),
            pltpu.VMEM((LANES, H), jnp.float32),
            pltpu.SemaphoreType.DMA,
        ],
    )


def _tc_body(x_ref, s_ref, trow_ref, out_ref):
    out_ref[...] = x_ref[...] + (s_ref[...] + trow_ref[0][None])[None]


def _tc_add(x, spatial, trow, bb):
    B, N, H = x.shape
    return pl.pallas_call(
        _tc_body,
        grid=(B // bb,),
        in_specs=[
            pl.BlockSpec((bb, N, H), lambda i: (i, 0, 0)),
            pl.BlockSpec((N, H), lambda i: (0, 0)),
            pl.BlockSpec((LANES, H), lambda i: (0, 0)),
        ],
        out_specs=pl.BlockSpec((bb, N, H), lambda i: (i, 0, 0)),
        out_shape=jax.ShapeDtypeStruct((B, N, H), jnp.float32),
        compiler_params=pltpu.CompilerParams(vmem_limit_bytes=100 * 1024 * 1024),
    )(x, spatial, trow)


def kernel(x, temporal_embed, spatial_embed, temporal_idx, num_patches):
    H = temporal_embed.shape[1]
    T = temporal_embed.shape[0]
    tidx = jnp.full((LANES,), temporal_idx, dtype=jnp.int32)
    trow = _make_sc_trow(H, T)(temporal_embed, tidx)
    return _tc_add(x, spatial_embed, trow, 8)


# speedup vs baseline: 1.0007x; 1.0007x over previous
import jax
import jax.numpy as jnp
from jax import lax
from jax.experimental import pallas as pl
from jax.experimental.pallas import tpu as pltpu
from jax.experimental.pallas import tpu_sc as plsc

NC = 2
NS = 16
LANES = 16


def _make_sc_trow(H, T):
    mesh = plsc.VectorSubcoreMesh(core_axis_name="c", subcore_axis_name="s",
                                  num_cores=NC, num_subcores=NS)

    def body(t_hbm, tidx_hbm, out_hbm, idx_v, trow_v, sem):
        wid = lax.axis_index("s") * NC + lax.axis_index("c")

        @pl.when(wid == 0)
        def _():
            pltpu.sync_copy(tidx_hbm, idx_v)
            pltpu.async_copy(t_hbm.at[idx_v], trow_v, sem).wait()
            pltpu.sync_copy(trow_v, out_hbm)

    return pl.kernel(
        body,
        out_type=jax.ShapeDtypeStruct((LANES, H), jnp.float32),
        mesh=mesh,
        compiler_params=pltpu.CompilerParams(use_tc_tiling_on_sc=False,
                                             needs_layout_passes=False),
        scratch_types=[
            pltpu.VMEM((LANES,), jnp.int32),
            pltpu.VMEM((LANES, H), jnp.float32),
            pltpu.SemaphoreType.DMA,
        ],
    )


def _tc_body(x_ref, s_ref, trow_ref, out_ref):
    out_ref[...] = x_ref[...] + (s_ref[...] + trow_ref[0][None])[None]


def _tc_add(x, spatial, trow, bb):
    B, N, H = x.shape
    return pl.pallas_call(
        _tc_body,
        grid=(B // bb,),
        in_specs=[
            pl.BlockSpec((bb, N, H), lambda i: (i, 0, 0)),
            pl.BlockSpec((N, H), lambda i: (0, 0)),
            pl.BlockSpec((LANES, H), lambda i: (0, 0)),
        ],
        out_specs=pl.BlockSpec((bb, N, H), lambda i: (i, 0, 0)),
        out_shape=jax.ShapeDtypeStruct((B, N, H), jnp.float32),
        compiler_params=pltpu.CompilerParams(vmem_limit_bytes=100 * 1024 * 1024),
    )(x, spatial, trow)


def kernel(x, temporal_embed, spatial_embed, temporal_idx, num_patches):
    H = temporal_embed.shape[1]
    T = temporal_embed.shape[0]
    tidx = jnp.full((LANES,), temporal_idx, dtype=jnp.int32)
    trow = _make_sc_trow(H, T)(temporal_embed, tidx)
    return _tc_add(x, spatial_embed, trow, 8)
